# Initial kernel scaffold; baseline (speedup 1.0000x reference)
import jax
import jax.numpy as jnp
from jax.experimental import pallas as pl


def kernel(x, top_scores, selected_experts_indices, num_tokens_per_expert):
    raise NotImplementedError("write your pallas kernel here")


# speedup vs baseline: 16.9801x; 16.9801x over previous
import functools

import jax
import jax.numpy as jnp
from jax.experimental import pallas as pl

_BLOCK_TOKENS = 1024


def _dispatch_combine_block(x_ref, scores_ref, out_ref):
    s = jnp.sum(scores_ref[...], axis=1, keepdims=True)
    out_ref[...] = x_ref[...] * s


@functools.partial(jax.jit, static_argnames=())
def _run(x, top_scores):
    num_tokens, dim = x.shape
    top_k = top_scores.shape[1]
    grid = (num_tokens // _BLOCK_TOKENS,)
    return pl.pallas_call(
        _dispatch_combine_block,
        grid=grid,
        in_specs=[
            pl.BlockSpec((_BLOCK_TOKENS, dim), lambda i: (i, 0)),
            pl.BlockSpec((_BLOCK_TOKENS, top_k), lambda i: (i, 0)),
        ],
        out_specs=pl.BlockSpec((_BLOCK_TOKENS, dim), lambda i: (i, 0)),
        out_shape=jax.ShapeDtypeStruct((num_tokens, dim), x.dtype),
    )(x, top_scores)


def kernel(x, top_scores, selected_experts_indices, num_tokens_per_expert):
    del selected_experts_indices, num_tokens_per_expert
    return _run(x, top_scores)


# speedup vs baseline: 17.3793x; 1.0235x over previous
import functools

import jax
import jax.numpy as jnp
from jax.experimental import pallas as pl

_BLOCK_TOKENS = 4096


def _dispatch_combine_block(x_ref, scores_ref, out_ref):
    s = jnp.sum(scores_ref[...], axis=1, keepdims=True)
    out_ref[...] = x_ref[...] * s


@functools.partial(jax.jit, static_argnames=())
def _run(x, top_scores):
    num_tokens, dim = x.shape
    top_k = top_scores.shape[1]
    grid = (num_tokens // _BLOCK_TOKENS,)
    return pl.pallas_call(
        _dispatch_combine_block,
        grid=grid,
        in_specs=[
            pl.BlockSpec((_BLOCK_TOKENS, dim), lambda i: (i, 0)),
            pl.BlockSpec((_BLOCK_TOKENS, top_k), lambda i: (i, 0)),
        ],
        out_specs=pl.BlockSpec((_BLOCK_TOKENS, dim), lambda i: (i, 0)),
        out_shape=jax.ShapeDtypeStruct((num_tokens, dim), x.dtype),
    )(x, top_scores)


def kernel(x, top_scores, selected_experts_indices, num_tokens_per_expert):
    del selected_experts_indices, num_tokens_per_expert
    return _run(x, top_scores)
